# baseline (speedup 1.0000x reference)
import functools

import jax
import jax.numpy as jnp
from jax import lax
from jax.experimental import pallas as pl
from jax.experimental.pallas import tpu as pltpu
from jax.experimental.pallas import tpu_sc as plsc

N = 10000
K = 16
IN_DIM = 256
C = 256
OUT_DIM = 512
R = N * K
EPS = 1e-5

NW = 32

CB = 8
NPAD = 10240
RPAD = NPAD * K
NT = 10112
CPW = NPAD // NW
NCHUNK = CPW // CB
ROWS_PER_CHUNK = CB * K
PER_W = RPAD // NW

def _sc_mesh():
    return plsc.VectorSubcoreMesh(core_axis_name="c", subcore_axis_name="s",
                                  num_cores=2, num_subcores=16)


def _sc_gather_hist(nbr_hbm, ptsT_hbm, zeros_hbm, ones_hbm,
                    nbp_hbm, cnt_hbm,
                    idx_v, px_v, py_v, pz_v, rows_v, ones_v, cnt_sh):
    c = lax.axis_index("c")
    s = lax.axis_index("s")
    wid = s * 2 + c
    base = wid * PER_W

    pltpu.sync_copy(nbr_hbm.at[pl.ds(base, PER_W)], idx_v)
    pltpu.sync_copy(ptsT_hbm.at[pl.ds(0, NT)], px_v)
    pltpu.sync_copy(ptsT_hbm.at[pl.ds(NT, NT)], py_v)
    pltpu.sync_copy(ptsT_hbm.at[pl.ds(2 * NT, NT)], pz_v)

    @pl.when(s == 0)
    def _():
        pltpu.sync_copy(zeros_hbm, cnt_sh)

    plsc.subcore_barrier()
    pltpu.sync_copy(ones_hbm, ones_v)
    pltpu.sync_copy(ones_v, cnt_sh.at[idx_v], add=True)

    zero16 = jnp.zeros((16,), jnp.float32)
    ii = lax.iota(jnp.int32, 16)

    def group(g, _):
        jv = g * 16
        nv = idx_v[pl.ds(jv, 16)]
        cid = lax.shift_right_logical(base + jv + ii, 4)
        nx = plsc.load_gather(px_v, [nv])
        ny = plsc.load_gather(py_v, [nv])
        nz = plsc.load_gather(pz_v, [nv])
        rx = plsc.load_gather(px_v, [cid]) - nx
        ry = plsc.load_gather(py_v, [cid]) - ny
        rz = plsc.load_gather(pz_v, [cid]) - nz
        d2 = rx * rx + ry * ry + rz * rz
        flat = (ii + jv) * 8
        plsc.store_scatter(rows_v, [flat], d2)
        plsc.store_scatter(rows_v, [flat + 1], rx)
        plsc.store_scatter(rows_v, [flat + 2], ry)
        plsc.store_scatter(rows_v, [flat + 3], rz)
        plsc.store_scatter(rows_v, [flat + 4], nx)
        plsc.store_scatter(rows_v, [flat + 5], ny)
        plsc.store_scatter(rows_v, [flat + 6], nz)
        plsc.store_scatter(rows_v, [flat + 7], zero16)
        return 0

    lax.fori_loop(0, PER_W // 16, group, 0)

    pltpu.sync_copy(rows_v, nbp_hbm.at[pl.ds(base * 8, PER_W * 8)])

    plsc.subcore_barrier()

    @pl.when(s == 0)
    def _():
        pltpu.sync_copy(cnt_sh, cnt_hbm.at[pl.ds(c * NT, NT)])


def _run_sc_gather_hist(nbr_pad, ptsT):
    zeros = jnp.zeros((NT,), jnp.float32)
    ones = jnp.ones((PER_W,), jnp.float32)
    kern = pl.kernel(
        _sc_gather_hist,
        out_type=(
            jax.ShapeDtypeStruct((RPAD * 8,), jnp.float32),
            jax.ShapeDtypeStruct((2 * NT,), jnp.float32),
        ),
        mesh=_sc_mesh(),
        compiler_params=pltpu.CompilerParams(needs_layout_passes=False),
        scratch_types=[
            pltpu.VMEM((PER_W,), jnp.int32),
            pltpu.VMEM((NT,), jnp.float32),
            pltpu.VMEM((NT,), jnp.float32),
            pltpu.VMEM((NT,), jnp.float32),
            pltpu.VMEM((PER_W * 8,), jnp.float32),
            pltpu.VMEM((PER_W,), jnp.float32),
            pltpu.MemorySpace.VMEM_SHARED((NT,), jnp.float32),
        ],
    )
    return kern(nbr_pad, ptsT, zeros, ones)


C32 = C // 2


def _acc_chunk(buf, slab):
    def center(cc, _):
        rb = cc * K
        for j in range(C32 // 16):
            sl = pl.ds(j * 16, 16)
            v = [plsc.bitcast(buf[rb + r, sl], jnp.bfloat16)
                 for r in range(K)]
            while len(v) > 1:
                v = [v[2 * t] + v[2 * t + 1] for t in range(len(v) // 2)]
            slab[cc, sl] = plsc.bitcast(v[0], jnp.int32)
        return 0

    lax.fori_loop(0, CB, center, 0)


def _sc_gather_pool(nbr_hbm, z_hbm, out_hbm,
                    idx0, idx1, buf0, buf1, slab0, slab1, sem0, sem1):
    c = lax.axis_index("c")
    s = lax.axis_index("s")
    wid = s * 2 + c
    fbase = wid * CPW * K
    cbase = wid * CPW

    def fire(g, idxb, buf, sem):
        pltpu.sync_copy(nbr_hbm.at[pl.ds(fbase + g * ROWS_PER_CHUNK,
                                         ROWS_PER_CHUNK)], idxb)
        return pltpu.async_copy(z_hbm.at[idxb], buf, sem)

    fire(0, idx0, buf0, sem0)

    def step(t, _):
        g0 = 2 * t
        fire(g0 + 1, idx1, buf1, sem1)
        pltpu.make_async_copy(z_hbm.at[idx0], buf0, sem0).wait()
        _acc_chunk(buf0, slab0)
        pltpu.sync_copy(slab0, out_hbm.at[pl.ds(cbase + g0 * CB, CB)])

        @pl.when(g0 + 2 < NCHUNK)
        def _():
            fire(g0 + 2, idx0, buf0, sem0)

        pltpu.make_async_copy(z_hbm.at[idx1], buf1, sem1).wait()
        _acc_chunk(buf1, slab1)
        pltpu.sync_copy(slab1, out_hbm.at[pl.ds(cbase + (g0 + 1) * CB, CB)])
        return 0

    lax.fori_loop(0, NCHUNK // 2, step, 0)


def _run_sc_gather_pool(nbr_pad_flat, z):
    kern = pl.kernel(
        _sc_gather_pool,
        out_type=jax.ShapeDtypeStruct((NPAD, C32), jnp.int32),
        mesh=_sc_mesh(),
        compiler_params=pltpu.CompilerParams(needs_layout_passes=False),
        scratch_types=[
            pltpu.VMEM((ROWS_PER_CHUNK,), jnp.int32),
            pltpu.VMEM((ROWS_PER_CHUNK,), jnp.int32),
            pltpu.VMEM((ROWS_PER_CHUNK, C32), jnp.int32),
            pltpu.VMEM((ROWS_PER_CHUNK, C32), jnp.int32),
            pltpu.VMEM((CB, C32), jnp.int32),
            pltpu.VMEM((CB, C32), jnp.int32),
            pltpu.SemaphoreType.DMA,
            pltpu.SemaphoreType.DMA,
        ],
    )
    return kern(nbr_pad_flat, z)


_BR = 1000
_BC = 80
_BP = _BC * K
_PREC = lax.Precision.HIGHEST
_PREC_SUM = lax.Precision.DEFAULT


def _tc_linstats_body(x_ref, w_ref, b_ref, c8_ref, y_ref, s1_ref, s2_ref):
    y = (jnp.dot(x_ref[...], w_ref[...], preferred_element_type=jnp.float32,
                 precision=_PREC) + b_ref[...])
    y_ref[...] = y

    @pl.when(pl.program_id(0) == 0)
    def _():
        s1_ref[...] = jnp.zeros_like(s1_ref)
        s2_ref[...] = jnp.zeros_like(s2_ref)

    dn0 = (((0,), (0,)), ((), ()))
    c8 = c8_ref[...]
    s1_ref[...] += lax.dot_general(c8, y, dn0,
                                   preferred_element_type=jnp.float32,
                                   precision=_PREC_SUM)
    s2_ref[...] += lax.dot_general(c8, y * y, dn0,
                                   preferred_element_type=jnp.float32,
                                   precision=_PREC_SUM)


def _run_tc_linstats(feats, wt, bias, counts8):
    return pl.pallas_call(
        _tc_linstats_body,
        grid=(N // _BR,),
        in_specs=[
            pl.BlockSpec((_BR, IN_DIM), lambda i: (i, 0)),
            pl.BlockSpec((IN_DIM, C), lambda i: (0, 0)),
            pl.BlockSpec((1, C), lambda i: (0, 0)),
            pl.BlockSpec((_BR, 8), lambda i: (i, 0)),
        ],
        out_specs=[
            pl.BlockSpec((_BR, C), lambda i: (i, 0)),
            pl.BlockSpec((8, C), lambda i: (0, 0)),
            pl.BlockSpec((8, C), lambda i: (0, 0)),
        ],
        out_shape=[
            jax.ShapeDtypeStruct((N, C), jnp.float32),
            jax.ShapeDtypeStruct((8, C), jnp.float32),
            jax.ShapeDtypeStruct((8, C), jnp.float32),
        ],
    )(feats, wt, bias, counts8)


def _tc_affine_relu_body(y_ref, sc_ref, sh_ref, o_ref):
    z = jnp.maximum(y_ref[...] * sc_ref[...] + sh_ref[...], 0.0)
    o_ref[...] = z.astype(jnp.bfloat16)


def _run_tc_affine_relu(y, scale, shift):
    return pl.pallas_call(
        _tc_affine_relu_body,
        grid=(N // _BR,),
        in_specs=[
            pl.BlockSpec((_BR, C), lambda i: (i, 0)),
            pl.BlockSpec((1, C), lambda i: (0, 0)),
            pl.BlockSpec((1, C), lambda i: (0, 0)),
        ],
        out_specs=pl.BlockSpec((_BR, C), lambda i: (i, 0)),
        out_shape=jax.ShapeDtypeStruct((N, C), jnp.bfloat16),
    )(y, scale, shift)


def _tc_posfeat_body(x_ref, m0_ref, e0_ref, e7_ref, bm_ref, g_ref):
    x = x_ref[...]
    dist = jnp.sqrt(x[:, 0:1])
    bm = x * m0_ref[...] + dist * e0_ref[...] + e7_ref[...]
    bm_ref[...] = bm

    @pl.when(pl.program_id(0) == 0)
    def _():
        g_ref[...] = jnp.zeros_like(g_ref)

    g_ref[...] += lax.dot_general(bm, bm, (((0,), (0,)), ((), ())),
                                  preferred_element_type=jnp.float32,
                                  precision=_PREC_SUM)


def _run_tc_posfeat(nbp8, m0, e0, e7):
    return pl.pallas_call(
        _tc_posfeat_body,
        grid=(N // _BC,),
        in_specs=[
            pl.BlockSpec((_BP, 8), lambda i: (i, 0)),
            pl.BlockSpec((1, 8), lambda i: (0, 0)),
            pl.BlockSpec((1, 8), lambda i: (0, 0)),
            pl.BlockSpec((1, 8), lambda i: (0, 0)),
        ],
        out_specs=[
            pl.BlockSpec((_BP, 8), lambda i: (i, 0)),
            pl.BlockSpec((8, 8), lambda i: (0, 0)),
        ],
        out_shape=[
            jax.ShapeDtypeStruct((R, 8), jnp.float32),
            jax.ShapeDtypeStruct((8, 8), jnp.float32),
        ],
    )(nbp8, m0, e0, e7)


def _tc_posapply_body(bm_ref, w_ref, o_ref):
    pre = jnp.dot(bm_ref[...], w_ref[...],
                  preferred_element_type=jnp.float32, precision=_PREC_SUM)
    z = jnp.maximum(pre, 0.0)
    o_ref[...] = jnp.sum(z.reshape(_BC, K, C), axis=1)


def _run_tc_posapply(bm, bw):
    return pl.pallas_call(
        _tc_posapply_body,
        grid=(N // _BC,),
        in_specs=[
            pl.BlockSpec((_BP, 8), lambda i: (i, 0)),
            pl.BlockSpec((8, C), lambda i: (0, 0)),
        ],
        out_specs=pl.BlockSpec((_BC, C), lambda i: (i, 0)),
        out_shape=jax.ShapeDtypeStruct((N, C), jnp.float32),
    )(bm, bw)


def kernel(points, neighbors, feats, pos_W, pos_b, feat_W, feat_b,
           pos_gamma, pos_beta, feat_gamma, feat_beta):
    nbr_pad = jnp.concatenate(
        [neighbors.reshape(-1),
         jnp.full((RPAD - R,), N, jnp.int32)])
    ptsT = jnp.pad(points.T, ((0, 0), (0, NT - N))).reshape(-1)

    nbp_flat, cnt_flat = _run_sc_gather_hist(nbr_pad, ptsT)
    cnt_part = cnt_flat.reshape(2, NT)
    nbp8 = nbp_flat.reshape(RPAD, 8)[:R]

    counts8 = jnp.pad(cnt_part[:, :N].T, ((0, 0), (0, 6)))

    y, s1, s2 = _run_tc_linstats(feats, feat_W.T, feat_b.reshape(1, C),
                                 counts8)
    mean = (s1[0] + s1[1]) / R
    var = (s2[0] + s2[1]) / R - mean * mean
    fscale = feat_gamma / jnp.sqrt(var + EPS)
    fshift = feat_beta - mean * fscale
    z = _run_tc_affine_relu(y, fscale.reshape(1, C), fshift.reshape(1, C))

    z_pad = jnp.pad(z, ((0, NT - N), (0, 0)))
    z32 = lax.bitcast_convert_type(z_pad.reshape(NT, C32, 2), jnp.int32)
    out32 = _run_sc_gather_pool(nbr_pad, z32)[:N]
    feat_out = lax.bitcast_convert_type(
        out32[..., None], jnp.bfloat16).reshape(N, C).astype(jnp.float32)

    m0 = jnp.zeros((1, 8), jnp.float32).at[0, 1:7].set(1.0)
    e0_8 = jnp.zeros((1, 8), jnp.float32).at[0, 0].set(1.0)
    e7_8 = jnp.zeros((1, 8), jnp.float32).at[0, 7].set(1.0)
    bm, g8 = _run_tc_posfeat(nbp8, m0, e0_8, e7_8)

    ell = jnp.zeros((16, 8), jnp.float32)
    ell = ell.at[0, 0].set(1.0).at[15, 7].set(1.0)
    for axis in range(3):
        ell = (ell.at[1 + axis, 1 + axis].set(1.0)
                  .at[4 + axis, 1 + axis].set(1.0)
                  .at[4 + axis, 4 + axis].set(1.0)
                  .at[7 + axis, 4 + axis].set(1.0))
    g = ell @ g8 @ ell.T

    wext = jnp.zeros((C, 16), jnp.float32)
    wext = wext.at[:, :10].set(pos_W).at[:, 15].set(pos_b)
    pmean = (wext @ g[:, 15]) / R
    pe2 = jnp.einsum("ci,ij,cj->c", wext, g, wext) / R
    pvar = pe2 - pmean * pmean
    pscale = pos_gamma / jnp.sqrt(pvar + EPS)
    pshift = pos_beta - pmean * pscale
    wfold = (wext * pscale[:, None]).T
    wfold = wfold.at[15, :].add(pshift)
    bw = ell.T @ wfold

    pos_out = _run_tc_posapply(bm, bw)

    return jnp.concatenate([feat_out, pos_out], axis=1)


# speedup vs baseline: 1.4395x; 1.4395x over previous
import functools

import jax
import jax.numpy as jnp
from jax import lax
from jax.experimental import pallas as pl
from jax.experimental.pallas import tpu as pltpu
from jax.experimental.pallas import tpu_sc as plsc

N = 10000
K = 16
IN_DIM = 256
C = 256
OUT_DIM = 512
R = N * K
EPS = 1e-5

NW = 32

CB = 8
NPAD = 10240
RPAD = NPAD * K
NT = 10112
CPW = NPAD // NW
NCHUNK = CPW // CB
ROWS_PER_CHUNK = CB * K
PER_W = RPAD // NW

def _sc_mesh():
    return plsc.VectorSubcoreMesh(core_axis_name="c", subcore_axis_name="s",
                                  num_cores=2, num_subcores=16)


def _sc_gather_hist(nbr_hbm, ptsT_hbm, zeros_hbm, ones_hbm,
                    nbp_hbm, cnt_hbm,
                    idx_v, px_v, py_v, pz_v, rows_v, ones_v, cnt_sh):
    c = lax.axis_index("c")
    s = lax.axis_index("s")
    wid = s * 2 + c
    base = wid * PER_W

    pltpu.sync_copy(nbr_hbm.at[pl.ds(base, PER_W)], idx_v)
    pltpu.sync_copy(ptsT_hbm.at[pl.ds(0, NT)], px_v)
    pltpu.sync_copy(ptsT_hbm.at[pl.ds(NT, NT)], py_v)
    pltpu.sync_copy(ptsT_hbm.at[pl.ds(2 * NT, NT)], pz_v)

    @pl.when(s == 0)
    def _():
        pltpu.sync_copy(zeros_hbm, cnt_sh)

    plsc.subcore_barrier()
    pltpu.sync_copy(ones_hbm, ones_v)
    pltpu.sync_copy(ones_v, cnt_sh.at[idx_v], add=True)

    zero16 = jnp.zeros((16,), jnp.float32)
    ii = lax.iota(jnp.int32, 16)

    def group(g, _):
        jv = g * 16
        nv = idx_v[pl.ds(jv, 16)]
        cid = lax.shift_right_logical(base + jv + ii, 4)
        nx = plsc.load_gather(px_v, [nv])
        ny = plsc.load_gather(py_v, [nv])
        nz = plsc.load_gather(pz_v, [nv])
        rx = plsc.load_gather(px_v, [cid]) - nx
        ry = plsc.load_gather(py_v, [cid]) - ny
        rz = plsc.load_gather(pz_v, [cid]) - nz
        d2 = rx * rx + ry * ry + rz * rz
        flat = (ii + jv) * 8
        plsc.store_scatter(rows_v, [flat], d2)
        plsc.store_scatter(rows_v, [flat + 1], rx)
        plsc.store_scatter(rows_v, [flat + 2], ry)
        plsc.store_scatter(rows_v, [flat + 3], rz)
        plsc.store_scatter(rows_v, [flat + 4], nx)
        plsc.store_scatter(rows_v, [flat + 5], ny)
        plsc.store_scatter(rows_v, [flat + 6], nz)
        plsc.store_scatter(rows_v, [flat + 7], zero16)
        return 0

    lax.fori_loop(0, PER_W // 16, group, 0)

    pltpu.sync_copy(rows_v, nbp_hbm.at[pl.ds(base * 8, PER_W * 8)])

    plsc.subcore_barrier()

    @pl.when(s == 0)
    def _():
        pltpu.sync_copy(cnt_sh, cnt_hbm.at[pl.ds(c * NT, NT)])


def _run_sc_gather_hist(nbr_pad, ptsT):
    zeros = jnp.zeros((NT,), jnp.float32)
    ones = jnp.ones((PER_W,), jnp.float32)
    kern = pl.kernel(
        _sc_gather_hist,
        out_type=(
            jax.ShapeDtypeStruct((RPAD * 8,), jnp.float32),
            jax.ShapeDtypeStruct((2 * NT,), jnp.float32),
        ),
        mesh=_sc_mesh(),
        compiler_params=pltpu.CompilerParams(needs_layout_passes=False),
        scratch_types=[
            pltpu.VMEM((PER_W,), jnp.int32),
            pltpu.VMEM((NT,), jnp.float32),
            pltpu.VMEM((NT,), jnp.float32),
            pltpu.VMEM((NT,), jnp.float32),
            pltpu.VMEM((PER_W * 8,), jnp.float32),
            pltpu.VMEM((PER_W,), jnp.float32),
            pltpu.MemorySpace.VMEM_SHARED((NT,), jnp.float32),
        ],
    )
    return kern(nbr_pad, ptsT, zeros, ones)


C32 = C // 2


def _acc_chunk(buf, slab):
    def center(cc, _):
        rb = cc * K
        for j in range(C32 // 16):
            sl = pl.ds(j * 16, 16)
            v = [plsc.bitcast(buf[rb + r, sl], jnp.bfloat16)
                 for r in range(K)]
            while len(v) > 1:
                v = [v[2 * t] + v[2 * t + 1] for t in range(len(v) // 2)]
            slab[cc, sl] = plsc.bitcast(v[0], jnp.int32)
        return 0

    lax.fori_loop(0, CB, center, 0)


def _sc_gather_pool(nbr_hbm, z_hbm, out_hbm,
                    idx0, idx1, buf0, buf1, slab0, slab1, sem0, sem1):
    c = lax.axis_index("c")
    s = lax.axis_index("s")
    wid = s * 2 + c
    fbase = wid * CPW * K
    cbase = wid * CPW

    def fire(g, idxb, buf, sem):
        pltpu.sync_copy(nbr_hbm.at[pl.ds(fbase + g * ROWS_PER_CHUNK,
                                         ROWS_PER_CHUNK)], idxb)
        return pltpu.async_copy(z_hbm.at[idxb], buf, sem)

    fire(0, idx0, buf0, sem0)

    def step(t, _):
        g0 = 2 * t
        fire(g0 + 1, idx1, buf1, sem1)
        pltpu.make_async_copy(z_hbm.at[idx0], buf0, sem0).wait()
        _acc_chunk(buf0, slab0)
        pltpu.sync_copy(slab0, out_hbm.at[pl.ds(cbase + g0 * CB, CB)])

        @pl.when(g0 + 2 < NCHUNK)
        def _():
            fire(g0 + 2, idx0, buf0, sem0)

        pltpu.make_async_copy(z_hbm.at[idx1], buf1, sem1).wait()
        _acc_chunk(buf1, slab1)
        pltpu.sync_copy(slab1, out_hbm.at[pl.ds(cbase + (g0 + 1) * CB, CB)])
        return 0

    lax.fori_loop(0, NCHUNK // 2, step, 0)


def _run_sc_gather_pool(nbr_pad_flat, z):
    kern = pl.kernel(
        _sc_gather_pool,
        out_type=jax.ShapeDtypeStruct((NPAD, C32), jnp.int32),
        mesh=_sc_mesh(),
        compiler_params=pltpu.CompilerParams(needs_layout_passes=False),
        scratch_types=[
            pltpu.VMEM((ROWS_PER_CHUNK,), jnp.int32),
            pltpu.VMEM((ROWS_PER_CHUNK,), jnp.int32),
            pltpu.VMEM((ROWS_PER_CHUNK, C32), jnp.int32),
            pltpu.VMEM((ROWS_PER_CHUNK, C32), jnp.int32),
            pltpu.VMEM((CB, C32), jnp.int32),
            pltpu.VMEM((CB, C32), jnp.int32),
            pltpu.SemaphoreType.DMA,
            pltpu.SemaphoreType.DMA,
        ],
    )
    return kern(nbr_pad_flat, z)


_BR = 1000
_BC = 80
_BP = _BC * K
_PREC = lax.Precision.HIGHEST
_PREC_SUM = lax.Precision.DEFAULT


def _tc_linstats_body(x_ref, w_ref, b_ref, c8_ref, y_ref, s1_ref, s2_ref):
    y = (jnp.dot(x_ref[...], w_ref[...], preferred_element_type=jnp.float32,
                 precision=_PREC) + b_ref[...])
    y_ref[...] = y

    @pl.when(pl.program_id(0) == 0)
    def _():
        s1_ref[...] = jnp.zeros_like(s1_ref)
        s2_ref[...] = jnp.zeros_like(s2_ref)

    dn0 = (((0,), (0,)), ((), ()))
    c8 = c8_ref[...]
    s1_ref[...] += lax.dot_general(c8, y, dn0,
                                   preferred_element_type=jnp.float32,
                                   precision=_PREC_SUM)
    s2_ref[...] += lax.dot_general(c8, y * y, dn0,
                                   preferred_element_type=jnp.float32,
                                   precision=_PREC_SUM)


def _run_tc_linstats(feats, wt, bias, counts8):
    return pl.pallas_call(
        _tc_linstats_body,
        grid=(N // _BR,),
        in_specs=[
            pl.BlockSpec((_BR, IN_DIM), lambda i: (i, 0)),
            pl.BlockSpec((IN_DIM, C), lambda i: (0, 0)),
            pl.BlockSpec((1, C), lambda i: (0, 0)),
            pl.BlockSpec((_BR, 8), lambda i: (i, 0)),
        ],
        out_specs=[
            pl.BlockSpec((_BR, C), lambda i: (i, 0)),
            pl.BlockSpec((8, C), lambda i: (0, 0)),
            pl.BlockSpec((8, C), lambda i: (0, 0)),
        ],
        out_shape=[
            jax.ShapeDtypeStruct((N, C), jnp.float32),
            jax.ShapeDtypeStruct((8, C), jnp.float32),
            jax.ShapeDtypeStruct((8, C), jnp.float32),
        ],
    )(feats, wt, bias, counts8)


def _tc_affine_relu_body(y_ref, sc_ref, sh_ref, o_ref):
    z = jnp.maximum(y_ref[...] * sc_ref[...] + sh_ref[...], 0.0)
    zb = z.astype(jnp.bfloat16)
    u = lax.bitcast_convert_type(zb, jnp.uint16).astype(jnp.int32)
    o_ref[...] = u[:, :C32] | (u[:, C32:] << 16)


def _run_tc_affine_relu(y, scale, shift):
    return pl.pallas_call(
        _tc_affine_relu_body,
        grid=(N // _BR,),
        in_specs=[
            pl.BlockSpec((_BR, C), lambda i: (i, 0)),
            pl.BlockSpec((1, C), lambda i: (0, 0)),
            pl.BlockSpec((1, C), lambda i: (0, 0)),
        ],
        out_specs=pl.BlockSpec((_BR, C32), lambda i: (i, 0)),
        out_shape=jax.ShapeDtypeStruct((N, C32), jnp.int32),
    )(y, scale, shift)


def _tc_posfeat_body(x_ref, m0_ref, e0_ref, e7_ref, bm_ref, g_ref):
    x = x_ref[...]
    dist = jnp.sqrt(x[:, 0:1])
    bm = x * m0_ref[...] + dist * e0_ref[...] + e7_ref[...]
    bm_ref[...] = bm

    @pl.when(pl.program_id(0) == 0)
    def _():
        g_ref[...] = jnp.zeros_like(g_ref)

    g_ref[...] += lax.dot_general(bm, bm, (((0,), (0,)), ((), ())),
                                  preferred_element_type=jnp.float32,
                                  precision=_PREC_SUM)


def _run_tc_posfeat(nbp8, m0, e0, e7):
    return pl.pallas_call(
        _tc_posfeat_body,
        grid=(N // _BC,),
        in_specs=[
            pl.BlockSpec((_BP, 8), lambda i: (i, 0)),
            pl.BlockSpec((1, 8), lambda i: (0, 0)),
            pl.BlockSpec((1, 8), lambda i: (0, 0)),
            pl.BlockSpec((1, 8), lambda i: (0, 0)),
        ],
        out_specs=[
            pl.BlockSpec((_BP, 8), lambda i: (i, 0)),
            pl.BlockSpec((8, 8), lambda i: (0, 0)),
        ],
        out_shape=[
            jax.ShapeDtypeStruct((R, 8), jnp.float32),
            jax.ShapeDtypeStruct((8, 8), jnp.float32),
        ],
    )(nbp8, m0, e0, e7)


def _tc_posapply_body(bm_ref, w_ref, o_ref):
    pre = jnp.dot(bm_ref[...], w_ref[...],
                  preferred_element_type=jnp.float32, precision=_PREC_SUM)
    z = jnp.maximum(pre, 0.0)
    o_ref[...] = jnp.sum(z.reshape(_BC, K, C), axis=1)


def _run_tc_posapply(bm, bw):
    return pl.pallas_call(
        _tc_posapply_body,
        grid=(N // _BC,),
        in_specs=[
            pl.BlockSpec((_BP, 8), lambda i: (i, 0)),
            pl.BlockSpec((8, C), lambda i: (0, 0)),
        ],
        out_specs=pl.BlockSpec((_BC, C), lambda i: (i, 0)),
        out_shape=jax.ShapeDtypeStruct((N, C), jnp.float32),
    )(bm, bw)


def kernel(points, neighbors, feats, pos_W, pos_b, feat_W, feat_b,
           pos_gamma, pos_beta, feat_gamma, feat_beta):
    nbr_pad = jnp.concatenate(
        [neighbors.reshape(-1),
         jnp.full((RPAD - R,), N, jnp.int32)])
    ptsT = jnp.pad(points.T, ((0, 0), (0, NT - N))).reshape(-1)

    nbp_flat, cnt_flat = _run_sc_gather_hist(nbr_pad, ptsT)
    cnt_part = cnt_flat.reshape(2, NT)
    nbp8 = nbp_flat.reshape(RPAD, 8)[:R]

    counts8 = jnp.pad(cnt_part[:, :N].T, ((0, 0), (0, 6)))

    y, s1, s2 = _run_tc_linstats(feats, feat_W.T, feat_b.reshape(1, C),
                                 counts8)
    mean = (s1[0] + s1[1]) / R
    var = (s2[0] + s2[1]) / R - mean * mean
    fscale = feat_gamma / jnp.sqrt(var + EPS)
    fshift = feat_beta - mean * fscale
    z = _run_tc_affine_relu(y, fscale.reshape(1, C), fshift.reshape(1, C))

    z32 = jnp.pad(z, ((0, NT - N), (0, 0)))
    out32 = _run_sc_gather_pool(nbr_pad, z32)[:N]
    lo = (out32 & 0xFFFF).astype(jnp.uint16)
    hi = ((out32 >> 16) & 0xFFFF).astype(jnp.uint16)
    feat_out = lax.bitcast_convert_type(
        jnp.concatenate([lo, hi], axis=1), jnp.bfloat16).astype(jnp.float32)

    m0 = jnp.zeros((1, 8), jnp.float32).at[0, 1:7].set(1.0)
    e0_8 = jnp.zeros((1, 8), jnp.float32).at[0, 0].set(1.0)
    e7_8 = jnp.zeros((1, 8), jnp.float32).at[0, 7].set(1.0)
    bm, g8 = _run_tc_posfeat(nbp8, m0, e0_8, e7_8)

    ell = jnp.zeros((16, 8), jnp.float32)
    ell = ell.at[0, 0].set(1.0).at[15, 7].set(1.0)
    for axis in range(3):
        ell = (ell.at[1 + axis, 1 + axis].set(1.0)
                  .at[4 + axis, 1 + axis].set(1.0)
                  .at[4 + axis, 4 + axis].set(1.0)
                  .at[7 + axis, 4 + axis].set(1.0))
    g = ell @ g8 @ ell.T

    wext = jnp.zeros((C, 16), jnp.float32)
    wext = wext.at[:, :10].set(pos_W).at[:, 15].set(pos_b)
    pmean = (wext @ g[:, 15]) / R
    pe2 = jnp.einsum("ci,ij,cj->c", wext, g, wext) / R
    pvar = pe2 - pmean * pmean
    pscale = pos_gamma / jnp.sqrt(pvar + EPS)
    pshift = pos_beta - pmean * pscale
    wfold = (wext * pscale[:, None]).T
    wfold = wfold.at[15, :].add(pshift)
    bw = ell.T @ wfold

    pos_out = _run_tc_posapply(bm, bw)

    return jnp.concatenate([feat_out, pos_out], axis=1)


# speedup vs baseline: 1.4414x; 1.0013x over previous
import functools

import jax
import jax.numpy as jnp
from jax import lax
from jax.experimental import pallas as pl
from jax.experimental.pallas import tpu as pltpu
from jax.experimental.pallas import tpu_sc as plsc

N = 10000
K = 16
IN_DIM = 256
C = 256
OUT_DIM = 512
R = N * K
EPS = 1e-5

NW = 32

CB = 8
NPAD = 10240
RPAD = NPAD * K
NT = 10112
CPW = NPAD // NW
NCHUNK = CPW // CB
ROWS_PER_CHUNK = CB * K
PER_W = RPAD // NW

def _sc_mesh():
    return plsc.VectorSubcoreMesh(core_axis_name="c", subcore_axis_name="s",
                                  num_cores=2, num_subcores=16)


def _sc_gather_hist(nbr_hbm, ptsT_hbm, zeros_hbm, ones_hbm,
                    nbp_hbm, cnt_hbm,
                    idx_v, px_v, py_v, pz_v, rows_v, ones_v, cnt_sh):
    c = lax.axis_index("c")
    s = lax.axis_index("s")
    wid = s * 2 + c
    base = wid * PER_W

    pltpu.sync_copy(nbr_hbm.at[pl.ds(base, PER_W)], idx_v)
    pltpu.sync_copy(ptsT_hbm.at[pl.ds(0, NT)], px_v)
    pltpu.sync_copy(ptsT_hbm.at[pl.ds(NT, NT)], py_v)
    pltpu.sync_copy(ptsT_hbm.at[pl.ds(2 * NT, NT)], pz_v)

    @pl.when(s == 0)
    def _():
        pltpu.sync_copy(zeros_hbm, cnt_sh)

    plsc.subcore_barrier()
    pltpu.sync_copy(ones_hbm, ones_v)
    pltpu.sync_copy(ones_v, cnt_sh.at[idx_v], add=True)

    zero16 = jnp.zeros((16,), jnp.float32)
    ii = lax.iota(jnp.int32, 16)

    def group(g, _):
        jv = g * 16
        nv = idx_v[pl.ds(jv, 16)]
        cid = lax.shift_right_logical(base + jv + ii, 4)
        nx = plsc.load_gather(px_v, [nv])
        ny = plsc.load_gather(py_v, [nv])
        nz = plsc.load_gather(pz_v, [nv])
        rx = plsc.load_gather(px_v, [cid]) - nx
        ry = plsc.load_gather(py_v, [cid]) - ny
        rz = plsc.load_gather(pz_v, [cid]) - nz
        d2 = rx * rx + ry * ry + rz * rz
        flat = (ii + jv) * 8
        plsc.store_scatter(rows_v, [flat], d2)
        plsc.store_scatter(rows_v, [flat + 1], rx)
        plsc.store_scatter(rows_v, [flat + 2], ry)
        plsc.store_scatter(rows_v, [flat + 3], rz)
        plsc.store_scatter(rows_v, [flat + 4], nx)
        plsc.store_scatter(rows_v, [flat + 5], ny)
        plsc.store_scatter(rows_v, [flat + 6], nz)
        plsc.store_scatter(rows_v, [flat + 7], zero16)
        return 0

    lax.fori_loop(0, PER_W // 16, group, 0)

    pltpu.sync_copy(rows_v, nbp_hbm.at[pl.ds(base * 8, PER_W * 8)])

    plsc.subcore_barrier()

    @pl.when(s == 0)
    def _():
        pltpu.sync_copy(cnt_sh, cnt_hbm.at[pl.ds(c * NT, NT)])


def _run_sc_gather_hist(nbr_pad, ptsT):
    zeros = jnp.zeros((NT,), jnp.float32)
    ones = jnp.ones((PER_W,), jnp.float32)
    kern = pl.kernel(
        _sc_gather_hist,
        out_type=(
            jax.ShapeDtypeStruct((RPAD * 8,), jnp.float32),
            jax.ShapeDtypeStruct((2 * NT,), jnp.float32),
        ),
        mesh=_sc_mesh(),
        compiler_params=pltpu.CompilerParams(needs_layout_passes=False),
        scratch_types=[
            pltpu.VMEM((PER_W,), jnp.int32),
            pltpu.VMEM((NT,), jnp.float32),
            pltpu.VMEM((NT,), jnp.float32),
            pltpu.VMEM((NT,), jnp.float32),
            pltpu.VMEM((PER_W * 8,), jnp.float32),
            pltpu.VMEM((PER_W,), jnp.float32),
            pltpu.MemorySpace.VMEM_SHARED((NT,), jnp.float32),
        ],
    )
    return kern(nbr_pad, ptsT, zeros, ones)


C32 = C // 2


def _acc_chunk(buf, slab):
    def center(cc, _):
        rb = cc * K
        for j in range(C32 // 16):
            sl = pl.ds(j * 16, 16)
            v = [plsc.bitcast(buf[rb + r, sl], jnp.bfloat16)
                 for r in range(K)]
            while len(v) > 1:
                v = [v[2 * t] + v[2 * t + 1] for t in range(len(v) // 2)]
            slab[cc, sl] = plsc.bitcast(v[0], jnp.int32)
        return 0

    lax.fori_loop(0, CB, center, 0)


def _sc_gather_pool(nbr_hbm, z_hbm, out_hbm,
                    idx_all, buf0, buf1, slab0, slab1, sem0, sem1):
    c = lax.axis_index("c")
    s = lax.axis_index("s")
    wid = s * 2 + c
    fbase = wid * CPW * K
    cbase = wid * CPW

    pltpu.sync_copy(nbr_hbm.at[pl.ds(fbase, CPW * K)], idx_all)

    def islice(g):
        return idx_all.at[pl.ds(g * ROWS_PER_CHUNK, ROWS_PER_CHUNK)]

    def fire(g, buf, sem):
        return pltpu.async_copy(z_hbm.at[islice(g)], buf, sem)

    fire(0, buf0, sem0)

    def step(t, _):
        g0 = 2 * t
        fire(g0 + 1, buf1, sem1)
        pltpu.make_async_copy(z_hbm.at[islice(g0)], buf0, sem0).wait()
        _acc_chunk(buf0, slab0)
        pltpu.sync_copy(slab0, out_hbm.at[pl.ds(cbase + g0 * CB, CB)])

        @pl.when(g0 + 2 < NCHUNK)
        def _():
            fire(g0 + 2, buf0, sem0)

        pltpu.make_async_copy(z_hbm.at[islice(g0 + 1)], buf1, sem1).wait()
        _acc_chunk(buf1, slab1)
        pltpu.sync_copy(slab1, out_hbm.at[pl.ds(cbase + (g0 + 1) * CB, CB)])
        return 0

    lax.fori_loop(0, NCHUNK // 2, step, 0)


def _run_sc_gather_pool(nbr_pad_flat, z):
    kern = pl.kernel(
        _sc_gather_pool,
        out_type=jax.ShapeDtypeStruct((NPAD, C32), jnp.int32),
        mesh=_sc_mesh(),
        compiler_params=pltpu.CompilerParams(needs_layout_passes=False),
        scratch_types=[
            pltpu.VMEM((CPW * K,), jnp.int32),
            pltpu.VMEM((ROWS_PER_CHUNK, C32), jnp.int32),
            pltpu.VMEM((ROWS_PER_CHUNK, C32), jnp.int32),
            pltpu.VMEM((CB, C32), jnp.int32),
            pltpu.VMEM((CB, C32), jnp.int32),
            pltpu.SemaphoreType.DMA,
            pltpu.SemaphoreType.DMA,
        ],
    )
    return kern(nbr_pad_flat, z)


_BR = 1000
_BC = 80
_BP = _BC * K
_PREC = lax.Precision.HIGHEST
_PREC_SUM = lax.Precision.DEFAULT


def _tc_linstats_body(x_ref, w_ref, b_ref, c8_ref, y_ref, s1_ref, s2_ref):
    y = (jnp.dot(x_ref[...], w_ref[...], preferred_element_type=jnp.float32,
                 precision=_PREC) + b_ref[...])
    y_ref[...] = y

    @pl.when(pl.program_id(0) == 0)
    def _():
        s1_ref[...] = jnp.zeros_like(s1_ref)
        s2_ref[...] = jnp.zeros_like(s2_ref)

    dn0 = (((0,), (0,)), ((), ()))
    c8 = c8_ref[...]
    s1_ref[...] += lax.dot_general(c8, y, dn0,
                                   preferred_element_type=jnp.float32,
                                   precision=_PREC_SUM)
    s2_ref[...] += lax.dot_general(c8, y * y, dn0,
                                   preferred_element_type=jnp.float32,
                                   precision=_PREC_SUM)


def _run_tc_linstats(feats, wt, bias, counts8):
    return pl.pallas_call(
        _tc_linstats_body,
        grid=(N // _BR,),
        in_specs=[
            pl.BlockSpec((_BR, IN_DIM), lambda i: (i, 0)),
            pl.BlockSpec((IN_DIM, C), lambda i: (0, 0)),
            pl.BlockSpec((1, C), lambda i: (0, 0)),
            pl.BlockSpec((_BR, 8), lambda i: (i, 0)),
        ],
        out_specs=[
            pl.BlockSpec((_BR, C), lambda i: (i, 0)),
            pl.BlockSpec((8, C), lambda i: (0, 0)),
            pl.BlockSpec((8, C), lambda i: (0, 0)),
        ],
        out_shape=[
            jax.ShapeDtypeStruct((N, C), jnp.float32),
            jax.ShapeDtypeStruct((8, C), jnp.float32),
            jax.ShapeDtypeStruct((8, C), jnp.float32),
        ],
    )(feats, wt, bias, counts8)


def _tc_affine_relu_body(y_ref, sc_ref, sh_ref, o_ref):
    z = jnp.maximum(y_ref[...] * sc_ref[...] + sh_ref[...], 0.0)
    zb = z.astype(jnp.bfloat16)
    u = lax.bitcast_convert_type(zb, jnp.uint16).astype(jnp.int32)
    o_ref[...] = u[:, :C32] | (u[:, C32:] << 16)


def _run_tc_affine_relu(y, scale, shift):
    return pl.pallas_call(
        _tc_affine_relu_body,
        grid=(N // _BR,),
        in_specs=[
            pl.BlockSpec((_BR, C), lambda i: (i, 0)),
            pl.BlockSpec((1, C), lambda i: (0, 0)),
            pl.BlockSpec((1, C), lambda i: (0, 0)),
        ],
        out_specs=pl.BlockSpec((_BR, C32), lambda i: (i, 0)),
        out_shape=jax.ShapeDtypeStruct((N, C32), jnp.int32),
    )(y, scale, shift)


def _tc_posfeat_body(x_ref, m0_ref, e0_ref, e7_ref, bm_ref, g_ref):
    x = x_ref[...]
    dist = jnp.sqrt(x[:, 0:1])
    bm = x * m0_ref[...] + dist * e0_ref[...] + e7_ref[...]
    bm_ref[...] = bm

    @pl.when(pl.program_id(0) == 0)
    def _():
        g_ref[...] = jnp.zeros_like(g_ref)

    g_ref[...] += lax.dot_general(bm, bm, (((0,), (0,)), ((), ())),
                                  preferred_element_type=jnp.float32,
                                  precision=_PREC_SUM)


def _run_tc_posfeat(nbp8, m0, e0, e7):
    return pl.pallas_call(
        _tc_posfeat_body,
        grid=(N // _BC,),
        in_specs=[
            pl.BlockSpec((_BP, 8), lambda i: (i, 0)),
            pl.BlockSpec((1, 8), lambda i: (0, 0)),
            pl.BlockSpec((1, 8), lambda i: (0, 0)),
            pl.BlockSpec((1, 8), lambda i: (0, 0)),
        ],
        out_specs=[
            pl.BlockSpec((_BP, 8), lambda i: (i, 0)),
            pl.BlockSpec((8, 8), lambda i: (0, 0)),
        ],
        out_shape=[
            jax.ShapeDtypeStruct((R, 8), jnp.float32),
            jax.ShapeDtypeStruct((8, 8), jnp.float32),
        ],
    )(nbp8, m0, e0, e7)


def _tc_posapply_body(bm_ref, w_ref, o_ref):
    pre = jnp.dot(bm_ref[...], w_ref[...],
                  preferred_element_type=jnp.float32, precision=_PREC_SUM)
    z = jnp.maximum(pre, 0.0)
    o_ref[...] = jnp.sum(z.reshape(_BC, K, C), axis=1)


def _run_tc_posapply(bm, bw):
    return pl.pallas_call(
        _tc_posapply_body,
        grid=(N // _BC,),
        in_specs=[
            pl.BlockSpec((_BP, 8), lambda i: (i, 0)),
            pl.BlockSpec((8, C), lambda i: (0, 0)),
        ],
        out_specs=pl.BlockSpec((_BC, C), lambda i: (i, 0)),
        out_shape=jax.ShapeDtypeStruct((N, C), jnp.float32),
    )(bm, bw)


def kernel(points, neighbors, feats, pos_W, pos_b, feat_W, feat_b,
           pos_gamma, pos_beta, feat_gamma, feat_beta):
    nbr_pad = jnp.concatenate(
        [neighbors.reshape(-1),
         jnp.full((RPAD - R,), N, jnp.int32)])
    ptsT = jnp.pad(points.T, ((0, 0), (0, NT - N))).reshape(-1)

    nbp_flat, cnt_flat = _run_sc_gather_hist(nbr_pad, ptsT)
    cnt_part = cnt_flat.reshape(2, NT)
    nbp8 = nbp_flat.reshape(RPAD, 8)[:R]

    counts8 = jnp.pad(cnt_part[:, :N].T, ((0, 0), (0, 6)))

    y, s1, s2 = _run_tc_linstats(feats, feat_W.T, feat_b.reshape(1, C),
                                 counts8)
    mean = (s1[0] + s1[1]) / R
    var = (s2[0] + s2[1]) / R - mean * mean
    fscale = feat_gamma / jnp.sqrt(var + EPS)
    fshift = feat_beta - mean * fscale
    z = _run_tc_affine_relu(y, fscale.reshape(1, C), fshift.reshape(1, C))

    z32 = jnp.pad(z, ((0, NT - N), (0, 0)))
    out32 = _run_sc_gather_pool(nbr_pad, z32)[:N]
    lo = (out32 & 0xFFFF).astype(jnp.uint16)
    hi = ((out32 >> 16) & 0xFFFF).astype(jnp.uint16)
    feat_out = lax.bitcast_convert_type(
        jnp.concatenate([lo, hi], axis=1), jnp.bfloat16).astype(jnp.float32)

    m0 = jnp.zeros((1, 8), jnp.float32).at[0, 1:7].set(1.0)
    e0_8 = jnp.zeros((1, 8), jnp.float32).at[0, 0].set(1.0)
    e7_8 = jnp.zeros((1, 8), jnp.float32).at[0, 7].set(1.0)
    bm, g8 = _run_tc_posfeat(nbp8, m0, e0_8, e7_8)

    ell = jnp.zeros((16, 8), jnp.float32)
    ell = ell.at[0, 0].set(1.0).at[15, 7].set(1.0)
    for axis in range(3):
        ell = (ell.at[1 + axis, 1 + axis].set(1.0)
                  .at[4 + axis, 1 + axis].set(1.0)
                  .at[4 + axis, 4 + axis].set(1.0)
                  .at[7 + axis, 4 + axis].set(1.0))
    g = ell @ g8 @ ell.T

    wext = jnp.zeros((C, 16), jnp.float32)
    wext = wext.at[:, :10].set(pos_W).at[:, 15].set(pos_b)
    pmean = (wext @ g[:, 15]) / R
    pe2 = jnp.einsum("ci,ij,cj->c", wext, g, wext) / R
    pvar = pe2 - pmean * pmean
    pscale = pos_gamma / jnp.sqrt(pvar + EPS)
    pshift = pos_beta - pmean * pscale
    wfold = (wext * pscale[:, None]).T
    wfold = wfold.at[15, :].add(pshift)
    bw = ell.T @ wfold

    pos_out = _run_tc_posapply(bm, bw)

    return jnp.concatenate([feat_out, pos_out], axis=1)
